# Initial kernel scaffold; baseline (speedup 1.0000x reference)
import jax
import jax.numpy as jnp
from jax.experimental import pallas as pl


def kernel(x):
    raise NotImplementedError("write your pallas kernel here")


# speedup vs baseline: 1.1332x; 1.1332x over previous
import functools

import jax
import jax.numpy as jnp
import numpy as np
from jax.experimental import pallas as pl
from jax.experimental.pallas import tpu as pltpu

_N = 32768
_Q = 256
_P = 128
_S = 64
_ROWS = 128
_R = 8
_TOPK = 5


def _dft_constants():
    t = np.arange(_Q, dtype=np.float64)
    q = np.arange(_Q, dtype=np.float64)
    p = np.arange(_P, dtype=np.float64)
    s = np.arange(_S, dtype=np.float64)
    a1 = 2.0 * np.pi * ((np.outer(t, q) % _Q) / _Q)
    e1re = np.cos(a1).astype(np.float32)
    e1im = (-np.sin(a1)).astype(np.float32)
    a2 = 2.0 * np.pi * ((np.outer(t, p) % _N) / _N)
    twre = np.cos(a2).astype(np.float32)
    twim = (-np.sin(a2)).astype(np.float32)
    a3 = 2.0 * np.pi * ((np.outer(p, s) % _P) / _P)
    f2re = np.cos(a3).astype(np.float32)
    f2im = (-np.sin(a3)).astype(np.float32)
    return e1re, e1im, twre, twim, f2re, f2im


def _mm(a, b):
    return jax.lax.dot_general(
        a, b, (((1,), (0,)), ((), ())),
        preferred_element_type=jnp.float32,
        precision=jax.lax.Precision.HIGHEST)


def _body(x_ref, e1re_ref, e1im_ref, twre_ref, twim_ref, f2re_ref, f2im_ref,
          season_ref, trend_ref):
    e1re = e1re_ref[...]
    e1im = e1im_ref[...]
    twre = twre_ref[...]
    twim = twim_ref[...]
    f2re = f2re_ref[...]
    f2im = f2im_ref[...]

    tio = jax.lax.broadcasted_iota(jnp.int32, (_Q, _S), 0).astype(jnp.float32)
    sio = jax.lax.broadcasted_iota(jnp.int32, (_Q, _S), 1).astype(jnp.float32)
    kidx = sio * 256.0 + tio
    qio = jax.lax.broadcasted_iota(jnp.int32, (_Q, 1), 0).astype(jnp.float32)
    pio = jax.lax.broadcasted_iota(jnp.int32, (1, _P), 1).astype(jnp.float32)
    altp = 1.0 - 2.0 * (pio - 2.0 * jnp.floor(pio * 0.5))

    two_pi = jnp.float32(2.0 * np.pi)
    inv_n = jnp.float32(1.0 / _N)

    for r in range(_R):
        x2 = x_ref[r]

        gre = _mm(e1re, x2)
        gim = _mm(e1im, x2)
        nyq = jnp.sum(gre[0:1, :] * altp)

        zre = gre * twre - gim * twim
        zim = gre * twim + gim * twre
        wre = _mm(zre, f2re) - _mm(zim, f2im)
        wim = _mm(zre, f2im) + _mm(zim, f2re)

        msq = wre * wre + wim * wim
        msq = jnp.where(kidx == 0.0, 0.0, msq)

        vals = []
        picks = []
        for _ in range(_TOPK):
            v = jnp.max(msq)
            pick = jnp.min(jnp.where(msq == v, kidx, jnp.float32(3.0e4)))
            vals.append(v)
            picks.append(pick)
            msq = jnp.where(kidx == pick, -1.0, msq)

        nyqsq = nyq * nyq
        thr = jnp.maximum(vals[4], jnp.minimum(vals[3], nyqsq))

        keepn = (nyqsq > thr).astype(jnp.float32)
        acc = (keepn * nyq * inv_n) * altp
        acc = jnp.broadcast_to(acc, (_Q, _P))

        for i in range(_TOPK - 1):
            keep = (vals[i] > thr).astype(jnp.float32)
            sel = (kidx == picks[i]).astype(jnp.float32)
            xre = jnp.sum(sel * wre)
            xim = jnp.sum(sel * wim)
            cre = 2.0 * inv_n * xre * keep
            cim = 2.0 * inv_n * xim * keep
            k = picks[i]
            kq = k * qio
            mq = kq - 256.0 * jnp.floor(kq * (1.0 / 256.0))
            aq = mq * (two_pi / 256.0)
            kp = k * pio
            mp = kp - 32768.0 * jnp.floor(kp * (1.0 / 32768.0))
            ap = mp * (two_pi / 32768.0)
            uqre = jnp.cos(aq)
            uqim = jnp.sin(aq)
            vpre = jnp.cos(ap)
            vpim = jnp.sin(ap)
            ure = cre * uqre - cim * uqim
            uim = cre * uqim + cim * uqre
            acc = acc + (ure * vpre - uim * vpim)

        season_ref[r] = acc
        trend_ref[r] = x2 - acc


@jax.jit
def kernel(x):
    e1re, e1im, twre, twim, f2re, f2im = _dft_constants()
    x3 = x.reshape(_ROWS, _Q, _P)
    grid = _ROWS // _R
    const_spec = lambda shp: pl.BlockSpec(shp, lambda i: (0, 0))
    season3, trend3 = pl.pallas_call(
        _body,
        grid=(grid,),
        in_specs=[
            pl.BlockSpec((_R, _Q, _P), lambda i: (i, 0, 0)),
            const_spec((_Q, _Q)), const_spec((_Q, _Q)),
            const_spec((_Q, _P)), const_spec((_Q, _P)),
            const_spec((_P, _S)), const_spec((_P, _S)),
        ],
        out_specs=[
            pl.BlockSpec((_R, _Q, _P), lambda i: (i, 0, 0)),
            pl.BlockSpec((_R, _Q, _P), lambda i: (i, 0, 0)),
        ],
        out_shape=[
            jax.ShapeDtypeStruct((_ROWS, _Q, _P), jnp.float32),
            jax.ShapeDtypeStruct((_ROWS, _Q, _P), jnp.float32),
        ],
    )(x3, jnp.asarray(e1re), jnp.asarray(e1im), jnp.asarray(twre),
      jnp.asarray(twim), jnp.asarray(f2re), jnp.asarray(f2im))
    return season3.reshape(_ROWS, _N), trend3.reshape(_ROWS, _N)


# speedup vs baseline: 1.9675x; 1.7363x over previous
import jax
import jax.numpy as jnp
import numpy as np
from jax.experimental import pallas as pl
from jax.experimental.pallas import tpu as pltpu

_N = 32768
_Q = 256
_P = 128
_S = 64
_ROWS = 128
_R = 8
_TOPK = 5


def _dft_constants():
    t = np.arange(_Q, dtype=np.float64)
    q = np.arange(_Q, dtype=np.float64)
    p = np.arange(_P, dtype=np.float64)
    s = np.arange(_S, dtype=np.float64)
    a1 = 2.0 * np.pi * ((np.outer(t, q) % _Q) / _Q)
    e1re = np.cos(a1).astype(np.float32)
    e1im = (-np.sin(a1)).astype(np.float32)
    a2 = 2.0 * np.pi * ((np.outer(t, p) % _N) / _N)
    twre = np.cos(a2).astype(np.float32)
    twim = (-np.sin(a2)).astype(np.float32)
    a3 = 2.0 * np.pi * ((np.outer(t, p) % _N) / _N)
    a4 = 2.0 * np.pi * ((np.outer(p, s) % _P) / _P)
    f2re = np.cos(a4).astype(np.float32)
    f2im = (-np.sin(a4)).astype(np.float32)
    a5 = 2.0 * np.pi * ((np.outer(s, p) % _P) / _P)
    b2re = np.cos(a5).astype(np.float32)
    b2im = np.sin(a5).astype(np.float32)
    return e1re, e1im, twre, twim, f2re, f2im, b2re, b2im


def _mm(a, b, prec=jax.lax.Precision.HIGHEST):
    return jax.lax.dot_general(
        a, b, (((1,), (0,)), ((), ())),
        preferred_element_type=jnp.float32, precision=prec)


def _reduce2(a, fn):
    return fn(fn(a, axis=1, keepdims=True), axis=2, keepdims=True)


def _body(x_ref, e1re_ref, e1im_ref, twre_ref, twim_ref, f2re_ref, f2im_ref,
          b2re_ref, b2im_ref, season_ref, trend_ref, msq_ref):
    e1re = e1re_ref[...]
    e1im = e1im_ref[...]
    twre = twre_ref[...]
    twim = twim_ref[...]
    f2re = f2re_ref[...]
    f2im = f2im_ref[...]
    b2re = b2re_ref[...]
    b2im = b2im_ref[...]

    gres = []
    gims = []
    for r in range(_R):
        x2 = x_ref[r]
        gres.append(_mm(e1re, x2))
        gims.append(_mm(e1im, x2))
    gre = jnp.stack(gres, axis=0)
    gim = jnp.stack(gims, axis=0)

    pio = jax.lax.broadcasted_iota(jnp.int32, (1, 1, _P), 2)
    altp = jnp.where((pio % 2) == 0, 1.0, -1.0).astype(jnp.float32)
    nyq = jnp.sum(gre[:, 0:1, :] * altp, axis=2, keepdims=True)

    zre = gre * twre - gim * twim
    zim = gre * twim + gim * twre
    zre2 = zre.reshape(_R * _Q, _P)
    zim2 = zim.reshape(_R * _Q, _P)
    wre = (_mm(zre2, f2re) - _mm(zim2, f2im)).reshape(_R, _Q, _S)
    wim = (_mm(zre2, f2im) + _mm(zim2, f2re)).reshape(_R, _Q, _S)

    tio = jax.lax.broadcasted_iota(jnp.int32, (_Q, _S), 0)
    sio = jax.lax.broadcasted_iota(jnp.int32, (_Q, _S), 1)
    bin0 = jnp.logical_and(tio == 0, sio == 0)
    nyqsq = nyq * nyq
    msq0 = jnp.where(bin0, nyqsq, wre * wre + wim * wim)

    msq_ref[...] = msq0
    msqm = msq_ref[...]

    msq = msqm
    vals = []
    cums = []
    cum = jnp.zeros((_R, 1, 1), jnp.float32)
    for _ in range(_TOPK):
        v = _reduce2(msq, jnp.max)
        eq = (msq == v)
        cnt = _reduce2(eq.astype(jnp.float32), jnp.sum)
        cum = cum + cnt
        vals.append(v)
        cums.append(cum)
        msq = jnp.where(eq, -1.0, msq)

    def nth(n):
        out = vals[_TOPK - 1]
        for j in range(_TOPK - 2, -1, -1):
            out = jnp.where(cums[j] >= n, vals[j], out)
        return out

    thr = nth(5.0)

    keep = msqm > thr * (1.0 + jnp.float32(2.0 ** -21))
    keepi = jnp.logical_and(keep, jnp.logical_not(bin0))
    xkre = jnp.where(keepi, wre, 0.0).reshape(_R * _Q, _S)
    xkim = jnp.where(keepi, wim, 0.0).reshape(_R * _Q, _S)
    hre = (_mm(xkre, b2re) - _mm(xkim, b2im)).reshape(_R, _Q, _P)
    him = (_mm(xkre, b2im) + _mm(xkim, b2re)).reshape(_R, _Q, _P)
    h2re = hre * twre + him * twim
    h2im = him * twre - hre * twim

    inv_n = jnp.float32(1.0 / _N)
    keepn = keep[:, 0:1, 0:1].astype(jnp.float32)
    nyqterm = (keepn * nyq * inv_n) * altp

    for r in range(_R):
        sea = 2.0 * inv_n * (_mm(e1re, h2re[r]) + _mm(e1im, h2im[r]))
        sea = sea + nyqterm[r]
        season_ref[r] = sea
        trend_ref[r] = x_ref[r] - sea


@jax.jit
def kernel(x):
    consts = _dft_constants()
    x3 = x.reshape(_ROWS, _Q, _P)
    grid = _ROWS // _R
    const_spec = lambda shp: pl.BlockSpec(shp, lambda i: (0, 0))
    season3, trend3 = pl.pallas_call(
        _body,
        grid=(grid,),
        in_specs=[
            pl.BlockSpec((_R, _Q, _P), lambda i: (i, 0, 0)),
            const_spec((_Q, _Q)), const_spec((_Q, _Q)),
            const_spec((_Q, _P)), const_spec((_Q, _P)),
            const_spec((_P, _S)), const_spec((_P, _S)),
            const_spec((_S, _P)), const_spec((_S, _P)),
        ],
        out_specs=[
            pl.BlockSpec((_R, _Q, _P), lambda i: (i, 0, 0)),
            pl.BlockSpec((_R, _Q, _P), lambda i: (i, 0, 0)),
        ],
        out_shape=[
            jax.ShapeDtypeStruct((_ROWS, _Q, _P), jnp.float32),
            jax.ShapeDtypeStruct((_ROWS, _Q, _P), jnp.float32),
        ],
        scratch_shapes=[pltpu.VMEM((_R, _Q, _S), jnp.float32)],
    )(x3, *[jnp.asarray(c) for c in consts])
    return season3.reshape(_ROWS, _N), trend3.reshape(_ROWS, _N)


# speedup vs baseline: 2.9602x; 1.5045x over previous
import jax
import jax.numpy as jnp
import numpy as np
from jax.experimental import pallas as pl
from jax.experimental.pallas import tpu as pltpu

_N = 32768
_Q = 256
_P = 128
_S = 64
_ROWS = 128
_R = 8
_TOPK = 5


def _dft_constants():
    t = np.arange(_Q, dtype=np.float64)
    q = np.arange(_Q, dtype=np.float64)
    p = np.arange(_P, dtype=np.float64)
    s = np.arange(_S, dtype=np.float64)
    a1 = 2.0 * np.pi * ((np.outer(t, q) % _Q) / _Q)
    e1re = np.cos(a1).astype(np.float32)
    e1im = (-np.sin(a1)).astype(np.float32)
    a2 = 2.0 * np.pi * ((np.outer(t, p) % _N) / _N)
    twre = np.cos(a2).astype(np.float32)
    twim = (-np.sin(a2)).astype(np.float32)
    a3 = 2.0 * np.pi * ((np.outer(t, p) % _N) / _N)
    a4 = 2.0 * np.pi * ((np.outer(p, s) % _P) / _P)
    f2re = np.cos(a4).astype(np.float32)
    f2im = (-np.sin(a4)).astype(np.float32)
    a5 = 2.0 * np.pi * ((np.outer(s, p) % _P) / _P)
    b2re = np.cos(a5).astype(np.float32)
    b2im = np.sin(a5).astype(np.float32)
    return e1re, e1im, twre, twim, f2re, f2im, b2re, b2im


def _mm(a, b, prec=jax.lax.Precision.HIGHEST):
    return jax.lax.dot_general(
        a, b, (((1,), (0,)), ((), ())),
        preferred_element_type=jnp.float32, precision=prec)


def _reduce2(a, fn):
    return fn(fn(a, axis=1, keepdims=True), axis=2, keepdims=True)


def _body(x_ref, e1re_ref, e1im_ref, twre_ref, twim_ref, f2re_ref, f2im_ref,
          b2re_ref, b2im_ref, season_ref, trend_ref, msq_ref):
    e1re = e1re_ref[...]
    e1im = e1im_ref[...]
    twre = twre_ref[...]
    twim = twim_ref[...]
    f2re = f2re_ref[...]
    f2im = f2im_ref[...]
    b2re = b2re_ref[...]
    b2im = b2im_ref[...]

    gres = []
    gims = []
    for r in range(_R):
        x2 = x_ref[r]
        gres.append(_mm(e1re, x2))
        gims.append(_mm(e1im, x2))
    gre = jnp.stack(gres, axis=0)
    gim = jnp.stack(gims, axis=0)

    pio = jax.lax.broadcasted_iota(jnp.int32, (1, 1, _P), 2)
    altp = jnp.where((pio % 2) == 0, 1.0, -1.0).astype(jnp.float32)
    nyq = jnp.sum(gre[:, 0:1, :] * altp, axis=2, keepdims=True)

    zre = gre * twre - gim * twim
    zim = gre * twim + gim * twre
    zre2 = zre.reshape(_R * _Q, _P)
    zim2 = zim.reshape(_R * _Q, _P)
    wre = (_mm(zre2, f2re) - _mm(zim2, f2im)).reshape(_R, _Q, _S)
    wim = (_mm(zre2, f2im) + _mm(zim2, f2re)).reshape(_R, _Q, _S)

    msq64 = wre * wre + wim * wim
    msq128 = jnp.concatenate(
        [msq64[:, 0:128, :], msq64[:, 128:256, :]], axis=2)
    tio = jax.lax.broadcasted_iota(jnp.int32, (128, 128), 0)
    sio = jax.lax.broadcasted_iota(jnp.int32, (128, 128), 1)
    bin0 = jnp.logical_and(tio == 0, sio == 0)
    nyqsq = nyq * nyq
    msq0 = jnp.where(bin0, nyqsq, msq128)

    msq_ref[...] = msq0
    msqm = msq_ref[...]

    msq = msqm
    vals = []
    cums = []
    cum = jnp.zeros((_R, 1, 1), jnp.float32)
    for _ in range(_TOPK):
        v = _reduce2(msq, jnp.max)
        eq = (msq == v)
        cnt = _reduce2(eq.astype(jnp.float32), jnp.sum)
        cum = cum + cnt
        vals.append(v)
        cums.append(cum)
        msq = jnp.where(eq, -1.0, msq)

    def nth(n):
        out = vals[_TOPK - 1]
        for j in range(_TOPK - 2, -1, -1):
            out = jnp.where(cums[j] >= n, vals[j], out)
        return out

    thr = nth(5.0)

    keep = msqm > thr * (1.0 + jnp.float32(2.0 ** -21))
    keepf = jnp.where(jnp.logical_and(keep, jnp.logical_not(bin0)),
                      1.0, 0.0).astype(jnp.float32)
    keepi = jnp.concatenate(
        [keepf[:, :, 0:_S], keepf[:, :, _S:2 * _S]], axis=1)
    xkre = (keepi * wre).reshape(_R * _Q, _S)
    xkim = (keepi * wim).reshape(_R * _Q, _S)
    hi = jax.lax.Precision.DEFAULT
    hre = (_mm(xkre, b2re, hi) - _mm(xkim, b2im, hi)).reshape(_R, _Q, _P)
    him = (_mm(xkre, b2im, hi) + _mm(xkim, b2re, hi)).reshape(_R, _Q, _P)
    h2re = hre * twre + him * twim
    h2im = him * twre - hre * twim

    inv_n = jnp.float32(1.0 / _N)
    keepn = keep[:, 0:1, 0:1].astype(jnp.float32)
    nyqterm = (keepn * nyq * inv_n) * altp

    for r in range(_R):
        sea = 2.0 * inv_n * (_mm(e1re, h2re[r], hi) + _mm(e1im, h2im[r], hi))
        sea = sea + nyqterm[r]
        season_ref[r] = sea
        trend_ref[r] = x_ref[r] - sea


@jax.jit
def kernel(x):
    consts = _dft_constants()
    x3 = x.reshape(_ROWS, _Q, _P)
    grid = _ROWS // _R
    const_spec = lambda shp: pl.BlockSpec(shp, lambda i: (0, 0))
    season3, trend3 = pl.pallas_call(
        _body,
        grid=(grid,),
        in_specs=[
            pl.BlockSpec((_R, _Q, _P), lambda i: (i, 0, 0)),
            const_spec((_Q, _Q)), const_spec((_Q, _Q)),
            const_spec((_Q, _P)), const_spec((_Q, _P)),
            const_spec((_P, _S)), const_spec((_P, _S)),
            const_spec((_S, _P)), const_spec((_S, _P)),
        ],
        out_specs=[
            pl.BlockSpec((_R, _Q, _P), lambda i: (i, 0, 0)),
            pl.BlockSpec((_R, _Q, _P), lambda i: (i, 0, 0)),
        ],
        out_shape=[
            jax.ShapeDtypeStruct((_ROWS, _Q, _P), jnp.float32),
            jax.ShapeDtypeStruct((_ROWS, _Q, _P), jnp.float32),
        ],
        scratch_shapes=[pltpu.VMEM((_R, _P, _P), jnp.float32)],
    )(x3, *[jnp.asarray(c) for c in consts])
    return season3.reshape(_ROWS, _N), trend3.reshape(_ROWS, _N)


# speedup vs baseline: 3.2126x; 1.0853x over previous
import jax
import jax.numpy as jnp
import numpy as np
from jax.experimental import pallas as pl
from jax.experimental.pallas import tpu as pltpu

_N = 32768
_Q = 256
_P = 128
_S = 64
_ROWS = 128
_R = 8
_TOPK = 5


def _dft_constants():
    t = np.arange(_Q, dtype=np.float64)
    q = np.arange(_Q, dtype=np.float64)
    p = np.arange(_P, dtype=np.float64)
    s = np.arange(_S, dtype=np.float64)
    a1 = 2.0 * np.pi * ((np.outer(t, q) % _Q) / _Q)
    e1re = np.cos(a1).astype(np.float32)
    e1im = (-np.sin(a1)).astype(np.float32)
    a2 = 2.0 * np.pi * ((np.outer(t, p) % _N) / _N)
    twre = np.cos(a2).astype(np.float32)
    twim = (-np.sin(a2)).astype(np.float32)
    a4 = 2.0 * np.pi * ((np.outer(p, s) % _P) / _P)
    f2re = np.cos(a4).astype(np.float32)
    f2im = (-np.sin(a4)).astype(np.float32)
    a5 = 2.0 * np.pi * ((np.outer(s, p) % _P) / _P)
    b2re = np.cos(a5).astype(np.float32)
    b2im = np.sin(a5).astype(np.float32)
    e1f = np.concatenate([e1re, e1im], axis=0)
    f2c = np.concatenate([f2re, f2im], axis=1)
    b2c = np.concatenate([b2re, b2im], axis=1)
    e1c = np.concatenate([e1re, e1im], axis=1)
    return e1f, twre, twim, f2c, b2c, e1c


def _mm(a, b, prec=jax.lax.Precision.HIGHEST):
    return jax.lax.dot_general(
        a, b, (((1,), (0,)), ((), ())),
        preferred_element_type=jnp.float32, precision=prec)


def _reduce2(a, fn):
    return fn(fn(a, axis=1, keepdims=True), axis=2, keepdims=True)


def _body(x_ref, e1f_ref, twre_ref, twim_ref, f2c_ref, b2c_ref, e1c_ref,
          season_ref, trend_ref, msq_ref):
    e1f = e1f_ref[...]
    twre = twre_ref[...]
    twim = twim_ref[...]
    f2c = f2c_ref[...]
    b2c = b2c_ref[...]
    e1c = e1c_ref[...]

    gres = []
    gims = []
    for r in range(_R):
        x2 = x_ref[r]
        g = _mm(e1f, x2)
        gres.append(g[0:_Q, :])
        gims.append(g[_Q:2 * _Q, :])
    gre = jnp.stack(gres, axis=0)
    gim = jnp.stack(gims, axis=0)

    pio = jax.lax.broadcasted_iota(jnp.int32, (1, 1, _P), 2)
    altp = jnp.where((pio % 2) == 0, 1.0, -1.0).astype(jnp.float32)
    nyq = jnp.sum(gre[:, 0:1, :] * altp, axis=2, keepdims=True)

    zre = gre * twre - gim * twim
    zim = gre * twim + gim * twre
    zcat = jnp.concatenate(
        [zre.reshape(_R * _Q, _P), zim.reshape(_R * _Q, _P)], axis=0)
    a2 = _mm(zcat, f2c)
    m = _R * _Q
    wre = (a2[0:m, 0:_S] - a2[m:2 * m, _S:2 * _S]).reshape(_R, _Q, _S)
    wim = (a2[0:m, _S:2 * _S] + a2[m:2 * m, 0:_S]).reshape(_R, _Q, _S)

    msq64 = wre * wre + wim * wim
    msq128 = jnp.concatenate(
        [msq64[:, 0:128, :], msq64[:, 128:256, :]], axis=2)
    tio = jax.lax.broadcasted_iota(jnp.int32, (128, 128), 0)
    sio = jax.lax.broadcasted_iota(jnp.int32, (128, 128), 1)
    bin0 = jnp.logical_and(tio == 0, sio == 0)
    nyqsq = nyq * nyq
    msq0 = jnp.where(bin0, nyqsq, msq128)

    msq_ref[...] = msq0
    msqm = msq_ref[...]

    msq = msqm
    vals = []
    cums = []
    cum = jnp.zeros((_R, 1, 1), jnp.float32)
    for _ in range(_TOPK):
        v = _reduce2(msq, jnp.max)
        eq = (msq == v)
        cnt = _reduce2(eq.astype(jnp.float32), jnp.sum)
        cum = cum + cnt
        vals.append(v)
        cums.append(cum)
        msq = jnp.where(eq, -1.0, msq)

    def nth(n):
        out = vals[_TOPK - 1]
        for j in range(_TOPK - 2, -1, -1):
            out = jnp.where(cums[j] >= n, vals[j], out)
        return out

    thr = nth(5.0)

    keep = msqm > thr * (1.0 + jnp.float32(2.0 ** -21))
    keepf = jnp.where(jnp.logical_and(keep, jnp.logical_not(bin0)),
                      1.0, 0.0).astype(jnp.float32)
    keepi = jnp.concatenate(
        [keepf[:, :, 0:_S], keepf[:, :, _S:2 * _S]], axis=1)
    xkre = (keepi * wre).reshape(_R * _Q, _S)
    xkim = (keepi * wim).reshape(_R * _Q, _S)
    hi = jax.lax.Precision.DEFAULT
    xcat = jnp.concatenate([xkre, xkim], axis=0)
    b = _mm(xcat, b2c, hi)
    hre = (b[0:m, 0:_P] - b[m:2 * m, _P:2 * _P]).reshape(_R, _Q, _P)
    him = (b[0:m, _P:2 * _P] + b[m:2 * m, 0:_P]).reshape(_R, _Q, _P)
    h2re = hre * twre + him * twim
    h2im = him * twre - hre * twim
    h2cat = jnp.concatenate([h2re, h2im], axis=1)

    inv_n = jnp.float32(1.0 / _N)
    keepn = keep[:, 0:1, 0:1].astype(jnp.float32)
    nyqterm = (keepn * nyq * inv_n) * altp

    for r in range(_R):
        sea = 2.0 * inv_n * _mm(e1c, h2cat[r], hi)
        sea = sea + nyqterm[r]
        season_ref[r] = sea
        trend_ref[r] = x_ref[r] - sea


@jax.jit
def kernel(x):
    consts = _dft_constants()
    x3 = x.reshape(_ROWS, _Q, _P)
    grid = _ROWS // _R
    const_spec = lambda shp: pl.BlockSpec(shp, lambda i: (0, 0))
    season3, trend3 = pl.pallas_call(
        _body,
        grid=(grid,),
        in_specs=[
            pl.BlockSpec((_R, _Q, _P), lambda i: (i, 0, 0)),
            const_spec((2 * _Q, _Q)),
            const_spec((_Q, _P)), const_spec((_Q, _P)),
            const_spec((_P, _P)),
            const_spec((_S, 2 * _P)),
            const_spec((_Q, 2 * _Q)),
        ],
        out_specs=[
            pl.BlockSpec((_R, _Q, _P), lambda i: (i, 0, 0)),
            pl.BlockSpec((_R, _Q, _P), lambda i: (i, 0, 0)),
        ],
        out_shape=[
            jax.ShapeDtypeStruct((_ROWS, _Q, _P), jnp.float32),
            jax.ShapeDtypeStruct((_ROWS, _Q, _P), jnp.float32),
        ],
        scratch_shapes=[pltpu.VMEM((_R, _P, _P), jnp.float32)],
    )(x3, *[jnp.asarray(c) for c in consts])
    return season3.reshape(_ROWS, _N), trend3.reshape(_ROWS, _N)


# speedup vs baseline: 3.3372x; 1.0388x over previous
import jax
import jax.numpy as jnp
import numpy as np
from jax.experimental import pallas as pl
from jax.experimental.pallas import tpu as pltpu

_N = 32768
_Q = 256
_P = 128
_S = 64
_ROWS = 128
_R = 16
_TOPK = 5


def _dft_constants():
    t = np.arange(_Q, dtype=np.float64)
    q = np.arange(_Q, dtype=np.float64)
    p = np.arange(_P, dtype=np.float64)
    s = np.arange(_S, dtype=np.float64)
    a1 = 2.0 * np.pi * ((np.outer(t, q) % _Q) / _Q)
    e1re = np.cos(a1).astype(np.float32)
    e1im = (-np.sin(a1)).astype(np.float32)
    a2 = 2.0 * np.pi * ((np.outer(t, p) % _N) / _N)
    twre = np.cos(a2).astype(np.float32)
    twim = (-np.sin(a2)).astype(np.float32)
    a4 = 2.0 * np.pi * ((np.outer(p, s) % _P) / _P)
    f2re = np.cos(a4).astype(np.float32)
    f2im = (-np.sin(a4)).astype(np.float32)
    a5 = 2.0 * np.pi * ((np.outer(s, p) % _P) / _P)
    b2re = np.cos(a5).astype(np.float32)
    b2im = np.sin(a5).astype(np.float32)
    e1f = np.concatenate([e1re, e1im], axis=0)
    f2c = np.concatenate([f2re, f2im], axis=1)
    b2c = np.concatenate([b2re, b2im], axis=1)
    e1c = np.concatenate([e1re, e1im], axis=1)
    return e1f, twre, twim, f2c, b2c, e1c


def _mm(a, b, prec=jax.lax.Precision.HIGHEST):
    return jax.lax.dot_general(
        a, b, (((1,), (0,)), ((), ())),
        preferred_element_type=jnp.float32, precision=prec)


def _reduce2(a, fn):
    return fn(fn(a, axis=1, keepdims=True), axis=2, keepdims=True)


def _body(x_ref, e1f_ref, twre_ref, twim_ref, f2c_ref, b2c_ref, e1c_ref,
          season_ref, trend_ref, msq_ref):
    e1f = e1f_ref[...]
    twre = twre_ref[...]
    twim = twim_ref[...]
    f2c = f2c_ref[...]
    b2c = b2c_ref[...]
    e1c = e1c_ref[...]

    gres = []
    gims = []
    for r in range(_R):
        x2 = x_ref[r]
        g = _mm(e1f, x2)
        gres.append(g[0:_Q, :])
        gims.append(g[_Q:2 * _Q, :])
    gre = jnp.stack(gres, axis=0)
    gim = jnp.stack(gims, axis=0)

    pio = jax.lax.broadcasted_iota(jnp.int32, (1, 1, _P), 2)
    altp = jnp.where((pio % 2) == 0, 1.0, -1.0).astype(jnp.float32)
    nyq = jnp.sum(gre[:, 0:1, :] * altp, axis=2, keepdims=True)

    zre = gre * twre - gim * twim
    zim = gre * twim + gim * twre
    zcat = jnp.concatenate(
        [zre.reshape(_R * _Q, _P), zim.reshape(_R * _Q, _P)], axis=0)
    a2 = _mm(zcat, f2c)
    m = _R * _Q
    wre = (a2[0:m, 0:_S] - a2[m:2 * m, _S:2 * _S]).reshape(_R, _Q, _S)
    wim = (a2[0:m, _S:2 * _S] + a2[m:2 * m, 0:_S]).reshape(_R, _Q, _S)

    msq64 = wre * wre + wim * wim
    msq128 = jnp.concatenate(
        [msq64[:, 0:128, :], msq64[:, 128:256, :]], axis=2)
    tio = jax.lax.broadcasted_iota(jnp.int32, (128, 128), 0)
    sio = jax.lax.broadcasted_iota(jnp.int32, (128, 128), 1)
    bin0 = jnp.logical_and(tio == 0, sio == 0)
    nyqsq = nyq * nyq
    msq0 = jnp.where(bin0, nyqsq, msq128)

    msq_ref[...] = msq0
    msqm = msq_ref[...]

    msq = msqm
    vals = []
    cums = []
    cum = jnp.zeros((_R, 1, 1), jnp.float32)
    for _ in range(_TOPK):
        v = _reduce2(msq, jnp.max)
        eq = (msq == v)
        cnt = _reduce2(eq.astype(jnp.float32), jnp.sum)
        cum = cum + cnt
        vals.append(v)
        cums.append(cum)
        msq = jnp.where(eq, -1.0, msq)

    def nth(n):
        out = vals[_TOPK - 1]
        for j in range(_TOPK - 2, -1, -1):
            out = jnp.where(cums[j] >= n, vals[j], out)
        return out

    thr = nth(5.0)

    keep = msqm > thr * (1.0 + jnp.float32(2.0 ** -21))
    keepf = jnp.where(jnp.logical_and(keep, jnp.logical_not(bin0)),
                      1.0, 0.0).astype(jnp.float32)
    keepi = jnp.concatenate(
        [keepf[:, :, 0:_S], keepf[:, :, _S:2 * _S]], axis=1)
    xkre = (keepi * wre).reshape(_R * _Q, _S)
    xkim = (keepi * wim).reshape(_R * _Q, _S)
    hi = jax.lax.Precision.DEFAULT
    xcat = jnp.concatenate([xkre, xkim], axis=0)
    b = _mm(xcat, b2c, hi)
    hre = (b[0:m, 0:_P] - b[m:2 * m, _P:2 * _P]).reshape(_R, _Q, _P)
    him = (b[0:m, _P:2 * _P] + b[m:2 * m, 0:_P]).reshape(_R, _Q, _P)
    h2re = hre * twre + him * twim
    h2im = him * twre - hre * twim
    h2cat = jnp.concatenate([h2re, h2im], axis=1)

    inv_n = jnp.float32(1.0 / _N)
    keepn = keep[:, 0:1, 0:1].astype(jnp.float32)
    nyqterm = (keepn * nyq * inv_n) * altp

    for r in range(_R):
        sea = 2.0 * inv_n * _mm(e1c, h2cat[r], hi)
        sea = sea + nyqterm[r]
        season_ref[r] = sea
        trend_ref[r] = x_ref[r] - sea


@jax.jit
def kernel(x):
    consts = _dft_constants()
    x3 = x.reshape(_ROWS, _Q, _P)
    grid = _ROWS // _R
    const_spec = lambda shp: pl.BlockSpec(shp, lambda i: (0, 0))
    season3, trend3 = pl.pallas_call(
        _body,
        grid=(grid,),
        in_specs=[
            pl.BlockSpec((_R, _Q, _P), lambda i: (i, 0, 0)),
            const_spec((2 * _Q, _Q)),
            const_spec((_Q, _P)), const_spec((_Q, _P)),
            const_spec((_P, _P)),
            const_spec((_S, 2 * _P)),
            const_spec((_Q, 2 * _Q)),
        ],
        out_specs=[
            pl.BlockSpec((_R, _Q, _P), lambda i: (i, 0, 0)),
            pl.BlockSpec((_R, _Q, _P), lambda i: (i, 0, 0)),
        ],
        out_shape=[
            jax.ShapeDtypeStruct((_ROWS, _Q, _P), jnp.float32),
            jax.ShapeDtypeStruct((_ROWS, _Q, _P), jnp.float32),
        ],
        scratch_shapes=[pltpu.VMEM((_R, _P, _P), jnp.float32)],
    )(x3, *[jnp.asarray(c) for c in consts])
    return season3.reshape(_ROWS, _N), trend3.reshape(_ROWS, _N)


# speedup vs baseline: 3.7450x; 1.1222x over previous
import jax
import jax.numpy as jnp
import numpy as np
from jax.experimental import pallas as pl
from jax.experimental.pallas import tpu as pltpu

_N = 32768
_Q = 256
_P = 128
_S = 64
_ROWS = 128
_R = 16
_TOPK = 5


def _dft_constants():
    t = np.arange(_Q, dtype=np.float64)
    q = np.arange(_Q, dtype=np.float64)
    p = np.arange(_P, dtype=np.float64)
    s = np.arange(_S, dtype=np.float64)
    a1 = 2.0 * np.pi * ((np.outer(t, q) % _Q) / _Q)
    e1re = np.cos(a1).astype(np.float32)
    e1im = (-np.sin(a1)).astype(np.float32)
    a2 = 2.0 * np.pi * ((np.outer(t, p) % _N) / _N)
    twre = np.cos(a2).astype(np.float32)
    twim = (-np.sin(a2)).astype(np.float32)
    a4 = 2.0 * np.pi * ((np.outer(p, s) % _P) / _P)
    f2re = np.cos(a4).astype(np.float32)
    f2im = (-np.sin(a4)).astype(np.float32)
    a5 = 2.0 * np.pi * ((np.outer(s, p) % _P) / _P)
    b2re = np.cos(a5).astype(np.float32)
    b2im = np.sin(a5).astype(np.float32)
    pi = np.concatenate([np.arange(128), [128], np.arange(255, 128, -1)])
    e1fh = np.concatenate([e1re[0:136], e1im[0:136]], axis=0)
    tw136re = twre[0:136]
    tw136im = twim[0:136]
    cp = 2.0 * np.pi * (p % _P) / _P
    cre = np.cos(cp).astype(np.float32)[None, :]
    cim = (-np.sin(cp)).astype(np.float32)[None, :]
    twre_p = twre[pi, :]
    twim_p = twim[pi, :]
    f2c = np.concatenate([f2re, f2im], axis=1)
    b2c = np.concatenate([b2re, b2im], axis=1)
    e1c_p = np.concatenate([e1re[:, pi], e1im[:, pi]], axis=1)
    return e1fh, tw136re, tw136im, cre, cim, twre_p, twim_p, f2c, b2c, e1c_p


def _mm(a, b, prec=jax.lax.Precision.HIGHEST):
    return jax.lax.dot_general(
        a, b, (((1,), (0,)), ((), ())),
        preferred_element_type=jnp.float32, precision=prec)


def _reduce2(a, fn):
    return fn(fn(a, axis=1, keepdims=True), axis=2, keepdims=True)


def _body(x_ref, e1fh_ref, tw136re_ref, tw136im_ref, cre_ref, cim_ref,
          twre_p_ref, twim_p_ref, f2c_ref, b2c_ref, e1c_ref,
          season_ref, trend_ref, msq_ref):
    e1fh = e1fh_ref[...]
    tw136re = tw136re_ref[...]
    tw136im = tw136im_ref[...]
    cre = cre_ref[...]
    cim = cim_ref[...]
    twre_p = twre_p_ref[...]
    twim_p = twim_p_ref[...]
    f2c = f2c_ref[...]
    b2c = b2c_ref[...]
    e1c = e1c_ref[...]

    rowj = jax.lax.broadcasted_iota(jnp.int32, (_P, _P), 0)
    zres = []
    zims = []
    g0s = []
    for r in range(_R):
        x2 = x_ref[r]
        g = _mm(e1fh, x2)
        g0s.append(g[0:1, :])
        zre_l = g[0:136, :] * tw136re - g[136:272, :] * tw136im
        zim_l = g[0:136, :] * tw136im + g[136:272, :] * tw136re
        ztop_re = zre_l[0:128, :]
        ztop_im = zim_l[0:128, :]
        zhi_re = ztop_re * cre + ztop_im * cim
        zhi_im = ztop_re * cim - ztop_im * cre
        zbot_re = jnp.where(rowj == 0, zre_l[128:129, :], zhi_re)
        zbot_im = jnp.where(rowj == 0, zim_l[128:129, :], zhi_im)
        zres.append(jnp.concatenate([ztop_re, zbot_re], axis=0))
        zims.append(jnp.concatenate([ztop_im, zbot_im], axis=0))
    zre = jnp.stack(zres, axis=0)
    zim = jnp.stack(zims, axis=0)

    pio = jax.lax.broadcasted_iota(jnp.int32, (1, 1, _P), 2)
    altp = jnp.where((pio % 2) == 0, 1.0, -1.0).astype(jnp.float32)
    g0 = jnp.stack(g0s, axis=0)
    nyq = jnp.sum(g0 * altp, axis=2, keepdims=True)

    zcat = jnp.concatenate(
        [zre.reshape(_R * _Q, _P), zim.reshape(_R * _Q, _P)], axis=0)
    a2 = _mm(zcat, f2c)
    m = _R * _Q
    wre = (a2[0:m, 0:_S] - a2[m:2 * m, _S:2 * _S]).reshape(_R, _Q, _S)
    wim = (a2[0:m, _S:2 * _S] + a2[m:2 * m, 0:_S]).reshape(_R, _Q, _S)

    msq64 = wre * wre + wim * wim
    msq128 = jnp.concatenate(
        [msq64[:, 0:128, :], msq64[:, 128:256, :]], axis=2)
    tio = jax.lax.broadcasted_iota(jnp.int32, (128, 128), 0)
    sio = jax.lax.broadcasted_iota(jnp.int32, (128, 128), 1)
    bin0 = jnp.logical_and(tio == 0, sio == 0)
    nyqsq = nyq * nyq
    msq0 = jnp.where(bin0, nyqsq, msq128)

    msq_ref[...] = msq0
    msqm = msq_ref[...]

    msq = msqm
    vals = []
    cums = []
    cum = jnp.zeros((_R, 1, 1), jnp.float32)
    for _ in range(_TOPK):
        v = _reduce2(msq, jnp.max)
        eq = (msq == v)
        cnt = _reduce2(eq.astype(jnp.float32), jnp.sum)
        cum = cum + cnt
        vals.append(v)
        cums.append(cum)
        msq = jnp.where(eq, -1.0, msq)

    def nth(n):
        out = vals[_TOPK - 1]
        for j in range(_TOPK - 2, -1, -1):
            out = jnp.where(cums[j] >= n, vals[j], out)
        return out

    thr = nth(5.0)

    keep = msqm > thr * (1.0 + jnp.float32(2.0 ** -21))
    keepf = jnp.where(jnp.logical_and(keep, jnp.logical_not(bin0)),
                      1.0, 0.0).astype(jnp.float32)
    keepi = jnp.concatenate(
        [keepf[:, :, 0:_S], keepf[:, :, _S:2 * _S]], axis=1)
    xkre = (keepi * wre).reshape(_R * _Q, _S)
    xkim = (keepi * wim).reshape(_R * _Q, _S)
    hi = jax.lax.Precision.DEFAULT
    xcat = jnp.concatenate([xkre, xkim], axis=0)
    b = _mm(xcat, b2c, hi)
    hre = (b[0:m, 0:_P] - b[m:2 * m, _P:2 * _P]).reshape(_R, _Q, _P)
    him = (b[0:m, _P:2 * _P] + b[m:2 * m, 0:_P]).reshape(_R, _Q, _P)
    h2re = hre * twre_p + him * twim_p
    h2im = him * twre_p - hre * twim_p
    h2cat = jnp.concatenate([h2re, h2im], axis=1)

    inv_n = jnp.float32(1.0 / _N)
    keepn = keep[:, 0:1, 0:1].astype(jnp.float32)
    nyqterm = (keepn * nyq * inv_n) * altp

    for r in range(_R):
        sea = 2.0 * inv_n * _mm(e1c, h2cat[r], hi)
        sea = sea + nyqterm[r]
        season_ref[r] = sea
        trend_ref[r] = x_ref[r] - sea


@jax.jit
def kernel(x):
    consts = _dft_constants()
    x3 = x.reshape(_ROWS, _Q, _P)
    grid = _ROWS // _R
    const_spec = lambda shp: pl.BlockSpec(shp, lambda i: (0, 0))
    season3, trend3 = pl.pallas_call(
        _body,
        grid=(grid,),
        in_specs=[
            pl.BlockSpec((_R, _Q, _P), lambda i: (i, 0, 0)),
            const_spec((272, _Q)),
            const_spec((136, _P)), const_spec((136, _P)),
            const_spec((1, _P)), const_spec((1, _P)),
            const_spec((_Q, _P)), const_spec((_Q, _P)),
            const_spec((_P, _P)),
            const_spec((_S, 2 * _P)),
            const_spec((_Q, 2 * _Q)),
        ],
        out_specs=[
            pl.BlockSpec((_R, _Q, _P), lambda i: (i, 0, 0)),
            pl.BlockSpec((_R, _Q, _P), lambda i: (i, 0, 0)),
        ],
        out_shape=[
            jax.ShapeDtypeStruct((_ROWS, _Q, _P), jnp.float32),
            jax.ShapeDtypeStruct((_ROWS, _Q, _P), jnp.float32),
        ],
        scratch_shapes=[pltpu.VMEM((_R, _P, _P), jnp.float32)],
    )(x3, *[jnp.asarray(c) for c in consts])
    return season3.reshape(_ROWS, _N), trend3.reshape(_ROWS, _N)


# speedup vs baseline: 4.8204x; 1.2872x over previous
import jax
import jax.numpy as jnp
import numpy as np
from jax.experimental import pallas as pl
from jax.experimental.pallas import tpu as pltpu

_N = 32768
_A = 128
_B = 256
_T = 128
_Sx = 128
_ROWS = 128
_R = 16
_TOPK = 5


def _dft_constants():
    t = np.arange(_T, dtype=np.float64)
    a = np.arange(_A, dtype=np.float64)
    b = np.arange(_B, dtype=np.float64)
    s = np.arange(_Sx, dtype=np.float64)
    a1 = 2.0 * np.pi * ((np.outer(t, a) % _A) / _A)
    e1re = np.cos(a1).astype(np.float32)
    e1im = (-np.sin(a1)).astype(np.float32)
    a2 = 2.0 * np.pi * ((np.outer(t, b) % _N) / _N)
    twre = np.cos(a2).astype(np.float32)
    twim = (-np.sin(a2)).astype(np.float32)
    a3 = 2.0 * np.pi * ((np.outer(b, s) % _B) / _B)
    f2re = np.cos(a3).astype(np.float32)
    f2im = (-np.sin(a3)).astype(np.float32)
    a4 = 2.0 * np.pi * ((np.outer(s, b) % _B) / _B)
    b2re = np.cos(a4).astype(np.float32)
    b2im = np.sin(a4).astype(np.float32)
    pi = np.concatenate([np.arange(64), [64], np.arange(127, 64, -1)])
    e1fh = np.concatenate([e1re[0:72], e1im[0:72]], axis=0)
    tw72re = twre[0:72]
    tw72im = twim[0:72]
    cb = 2.0 * np.pi * (b % _B) / _B
    cre = np.cos(cb).astype(np.float32)[None, :]
    cim = (-np.sin(cb)).astype(np.float32)[None, :]
    twre_p = twre[pi, :]
    twim_p = twim[pi, :]
    f2c = np.concatenate([f2re, f2im], axis=1)
    b2c = np.concatenate([b2re, b2im], axis=1)
    e1c_p = np.concatenate([e1re[:, pi], e1im[:, pi]], axis=1)
    return e1fh, tw72re, tw72im, cre, cim, twre_p, twim_p, f2c, b2c, e1c_p


def _mm(a, b, prec=jax.lax.Precision.HIGHEST):
    return jax.lax.dot_general(
        a, b, (((1,), (0,)), ((), ())),
        preferred_element_type=jnp.float32, precision=prec)


def _reduce2(x, fn):
    return fn(fn(x, axis=1, keepdims=True), axis=2, keepdims=True)


def _body(x_ref, e1fh_ref, tw72re_ref, tw72im_ref, cre_ref, cim_ref,
          twre_p_ref, twim_p_ref, f2c_ref, b2c_ref, e1c_ref,
          season_ref, trend_ref, msq_ref):
    e1fh = e1fh_ref[...]
    tw72re = tw72re_ref[...]
    tw72im = tw72im_ref[...]
    cre = cre_ref[...]
    cim = cim_ref[...]
    twre_p = twre_p_ref[...]
    twim_p = twim_p_ref[...]
    f2c = f2c_ref[...]
    b2c = b2c_ref[...]
    e1c = e1c_ref[...]

    rowj = jax.lax.broadcasted_iota(jnp.int32, (64, _B), 0)
    zres = []
    zims = []
    g0s = []
    for r in range(_R):
        x2 = x_ref[r]
        g = _mm(e1fh, x2)
        g0s.append(g[0:1, :])
        zre_l = g[0:72, :] * tw72re - g[72:144, :] * tw72im
        zim_l = g[0:72, :] * tw72im + g[72:144, :] * tw72re
        ztop_re = zre_l[0:64, :]
        ztop_im = zim_l[0:64, :]
        zhi_re = ztop_re * cre + ztop_im * cim
        zhi_im = ztop_re * cim - ztop_im * cre
        zbot_re = jnp.where(rowj == 0, zre_l[64:65, :], zhi_re)
        zbot_im = jnp.where(rowj == 0, zim_l[64:65, :], zhi_im)
        zres.append(jnp.concatenate([ztop_re, zbot_re], axis=0))
        zims.append(jnp.concatenate([ztop_im, zbot_im], axis=0))
    zre = jnp.stack(zres, axis=0)
    zim = jnp.stack(zims, axis=0)

    bio = jax.lax.broadcasted_iota(jnp.int32, (1, 1, _B), 2)
    altb = jnp.where((bio % 2) == 0, 1.0, -1.0).astype(jnp.float32)
    g0 = jnp.stack(g0s, axis=0)
    nyq = jnp.sum(g0 * altb, axis=2, keepdims=True)

    m = _R * _T
    zcat = jnp.concatenate(
        [zre.reshape(m, _B), zim.reshape(m, _B)], axis=0)
    a2 = _mm(zcat, f2c)
    wre = (a2[0:m, 0:_Sx] - a2[m:2 * m, _Sx:2 * _Sx]).reshape(_R, _T, _Sx)
    wim = (a2[0:m, _Sx:2 * _Sx] + a2[m:2 * m, 0:_Sx]).reshape(_R, _T, _Sx)

    tio = jax.lax.broadcasted_iota(jnp.int32, (_T, _Sx), 0)
    sio = jax.lax.broadcasted_iota(jnp.int32, (_T, _Sx), 1)
    bin0 = jnp.logical_and(tio == 0, sio == 0)
    nyqsq = nyq * nyq
    msq0 = jnp.where(bin0, nyqsq, wre * wre + wim * wim)

    msq_ref[...] = msq0
    msqm = msq_ref[...]

    msq = msqm
    vals = []
    cums = []
    cum = jnp.zeros((_R, 1, 1), jnp.float32)
    for _ in range(_TOPK):
        v = _reduce2(msq, jnp.max)
        eq = (msq == v)
        cnt = _reduce2(eq.astype(jnp.float32), jnp.sum)
        cum = cum + cnt
        vals.append(v)
        cums.append(cum)
        msq = jnp.where(eq, -1.0, msq)

    def nth(n):
        out = vals[_TOPK - 1]
        for j in range(_TOPK - 2, -1, -1):
            out = jnp.where(cums[j] >= n, vals[j], out)
        return out

    thr = nth(5.0)

    keep = msqm > thr * (1.0 + jnp.float32(2.0 ** -21))
    keepi = jnp.where(jnp.logical_and(keep, jnp.logical_not(bin0)),
                      1.0, 0.0).astype(jnp.float32)
    xkre = (keepi * wre).reshape(m, _Sx)
    xkim = (keepi * wim).reshape(m, _Sx)
    lo = jax.lax.Precision.DEFAULT
    xcat = jnp.concatenate([xkre, xkim], axis=0)
    bm = _mm(xcat, b2c, lo)
    hre = (bm[0:m, 0:_B] - bm[m:2 * m, _B:2 * _B]).reshape(_R, _T, _B)
    him = (bm[0:m, _B:2 * _B] + bm[m:2 * m, 0:_B]).reshape(_R, _T, _B)
    h2re = hre * twre_p + him * twim_p
    h2im = him * twre_p - hre * twim_p
    h2cat = jnp.concatenate([h2re, h2im], axis=1)

    inv_n = jnp.float32(1.0 / _N)
    keepn = keep[:, 0:1, 0:1].astype(jnp.float32)
    nyqterm = (keepn * nyq * inv_n) * altb

    for r in range(_R):
        sea = 2.0 * inv_n * _mm(e1c, h2cat[r], lo)
        sea = sea + nyqterm[r]
        season_ref[r] = sea
        trend_ref[r] = x_ref[r] - sea


@jax.jit
def kernel(x):
    consts = _dft_constants()
    x3 = x.reshape(_ROWS, _A, _B)
    grid = _ROWS // _R
    const_spec = lambda shp: pl.BlockSpec(shp, lambda i: (0, 0))
    season3, trend3 = pl.pallas_call(
        _body,
        grid=(grid,),
        in_specs=[
            pl.BlockSpec((_R, _A, _B), lambda i: (i, 0, 0)),
            const_spec((144, _A)),
            const_spec((72, _B)), const_spec((72, _B)),
            const_spec((1, _B)), const_spec((1, _B)),
            const_spec((_T, _B)), const_spec((_T, _B)),
            const_spec((_B, _B)),
            const_spec((_T, 2 * _B)),
            const_spec((_A, _B)),
        ],
        out_specs=[
            pl.BlockSpec((_R, _A, _B), lambda i: (i, 0, 0)),
            pl.BlockSpec((_R, _A, _B), lambda i: (i, 0, 0)),
        ],
        out_shape=[
            jax.ShapeDtypeStruct((_ROWS, _A, _B), jnp.float32),
            jax.ShapeDtypeStruct((_ROWS, _A, _B), jnp.float32),
        ],
        scratch_shapes=[pltpu.VMEM((_R, _T, _Sx), jnp.float32)],
    )(x3, *[jnp.asarray(c) for c in consts])
    return season3.reshape(_ROWS, _N), trend3.reshape(_ROWS, _N)
